# initial kernel scaffold (unmeasured)
import jax
import jax.numpy as jnp
from jax import lax
from jax.experimental import pallas as pl
from jax.experimental.pallas import tpu as pltpu

N_DEV = 16


def kernel(x, router_W, route_idx, expert_W, shared_W):
    n, d = x.shape
    e_per, _, h = expert_W.shape
    n_exp = router_W.shape[1]

    def body(x_ref, rw_ref, idx_ref, ew_ref, sw_ref, out_ref,
             comm_ref, send_sems, recv_sems):
        my = lax.axis_index("i")
        left = lax.rem(my - 1 + N_DEV, N_DEV)
        right = lax.rem(my + 1, N_DEV)

        barrier_sem = pltpu.get_barrier_semaphore()
        for nbr in (left, right):
            pl.semaphore_signal(
                barrier_sem, inc=1,
                device_id=(nbr,), device_id_type=pl.DeviceIdType.MESH,
            )
        pl.semaphore_wait(barrier_sem, 2)

        xf = x_ref[...]
        scores = jnp.dot(xf, rw_ref[...], preferred_element_type=jnp.float32)
        s_max = jnp.max(scores, axis=-1, keepdims=True)
        e = jnp.exp(scores - s_max)
        probs = e / jnp.sum(e, axis=-1, keepdims=True)
        idx = idx_ref[...]
        eids = lax.broadcasted_iota(jnp.int32, (n, n_exp), 1)
        p_sel = jnp.sum(jnp.where(eids == idx, probs, 0.0), axis=-1,
                        keepdims=True)

        acc = jnp.zeros((n, h), jnp.float32)
        for k in range(e_per):
            w = jnp.where(idx == my * e_per + k, p_sel, 0.0)
            xk = (xf * w).astype(jnp.bfloat16)
            acc = acc + jnp.dot(xk, ew_ref[k].astype(jnp.bfloat16),
                                preferred_element_type=jnp.float32)

        comm_ref[0, :, :] = acc.astype(jnp.bfloat16)
        out_ref[...] = acc

        for hop in range(N_DEV - 1):
            s = hop % 2
            r = (hop + 1) % 2
            rdma = pltpu.make_async_remote_copy(
                src_ref=comm_ref.at[s],
                dst_ref=comm_ref.at[r],
                send_sem=send_sems.at[s],
                recv_sem=recv_sems.at[r],
                device_id=(right,),
                device_id_type=pl.DeviceIdType.MESH,
            )
            rdma.start()
            rdma.wait()
            out_ref[...] = out_ref[...] + comm_ref[r, :, :].astype(jnp.float32)

        out_ref[...] = out_ref[...] + jnp.dot(
            xf.astype(jnp.bfloat16), sw_ref[...].astype(jnp.bfloat16),
            preferred_element_type=jnp.float32)

    return pl.pallas_call(
        body,
        out_shape=jax.ShapeDtypeStruct((n, h), jnp.float32),
        in_specs=[pl.BlockSpec(memory_space=pltpu.VMEM)] * 5,
        out_specs=pl.BlockSpec(memory_space=pltpu.VMEM),
        scratch_shapes=[
            pltpu.VMEM((2, n, h), jnp.bfloat16),
            pltpu.SemaphoreType.DMA((2,)),
            pltpu.SemaphoreType.DMA((2,)),
        ],
        compiler_params=pltpu.CompilerParams(collective_id=0),
    )(x, router_W, route_idx, expert_W, shared_W)


# baseline (device time: 748550 ns/iter reference)
import jax
import jax.numpy as jnp
from jax import lax
from jax.experimental import pallas as pl
from jax.experimental.pallas import tpu as pltpu

N_DEV = 16


def kernel(x, router_W, route_idx, expert_W, shared_W):
    n, d = x.shape
    e_per, _, h = expert_W.shape
    n_exp = router_W.shape[1]

    def body(x_ref, rw_ref, idx_ref, ew_ref, sw_ref, out_ref,
             comm_ref, send_sems, recv_sems):
        my = lax.axis_index("i")
        left = lax.rem(my - 1 + N_DEV, N_DEV)
        right = lax.rem(my + 1, N_DEV)

        barrier_sem = pltpu.get_barrier_semaphore()
        for nbr in (left, right):
            pl.semaphore_signal(
                barrier_sem, inc=1,
                device_id=(nbr,), device_id_type=pl.DeviceIdType.MESH,
            )
        pl.semaphore_wait(barrier_sem, 2)

        xf = x_ref[...]
        scores = jnp.dot(xf, rw_ref[...], preferred_element_type=jnp.float32)
        s_max = jnp.max(scores, axis=-1, keepdims=True)
        e = jnp.exp(scores - s_max)
        probs = e / jnp.sum(e, axis=-1, keepdims=True)
        idx = idx_ref[...]
        eids = lax.broadcasted_iota(jnp.int32, (n, n_exp), 1)
        p_sel = jnp.sum(jnp.where(eids == idx, probs, 0.0), axis=-1,
                        keepdims=True)

        acc = jnp.zeros((n, h), jnp.float32)
        for k in range(e_per):
            w = jnp.where(idx == my * e_per + k, p_sel, 0.0)
            xk = (xf * w).astype(jnp.bfloat16)
            acc = acc + jnp.dot(xk, ew_ref[k].astype(jnp.bfloat16),
                                preferred_element_type=jnp.float32)

        comm_ref[0, :, :] = acc.astype(jnp.bfloat16)
        out_ref[...] = acc

        for hop in range(N_DEV - 1):
            s = hop % 2
            r = (hop + 1) % 2
            rdma = pltpu.make_async_remote_copy(
                src_ref=comm_ref.at[s],
                dst_ref=comm_ref.at[r],
                send_sem=send_sems.at[s],
                recv_sem=recv_sems.at[r],
                device_id=(right,),
                device_id_type=pl.DeviceIdType.MESH,
            )
            rdma.start()
            rdma.wait()
            out_ref[...] = out_ref[...] + comm_ref[r, :, :].astype(jnp.float32)

        out_ref[...] = out_ref[...] + jnp.dot(
            xf.astype(jnp.bfloat16), sw_ref[...].astype(jnp.bfloat16),
            preferred_element_type=jnp.float32)

    return pl.pallas_call(
        body,
        out_shape=jax.ShapeDtypeStruct((n, h), jnp.float32),
        in_specs=[pl.BlockSpec(memory_space=pltpu.VMEM)] * 5,
        out_specs=pl.BlockSpec(memory_space=pltpu.VMEM),
        scratch_shapes=[
            pltpu.VMEM((2, n, h), jnp.bfloat16),
            pltpu.SemaphoreType.DMA((2,)),
            pltpu.SemaphoreType.DMA((2,)),
        ],
        compiler_params=pltpu.CompilerParams(
            collective_id=0,
            vmem_limit_bytes=96 * 1024 * 1024,
        ),
    )(x, router_W, route_idx, expert_W, shared_W)


# device time: 177603 ns/iter; 4.2147x vs baseline; 4.2147x over previous
import jax
import jax.numpy as jnp
from jax import lax
from jax.experimental import pallas as pl
from jax.experimental.pallas import tpu as pltpu

N_DEV = 16


def kernel(x, router_W, route_idx, expert_W, shared_W):
    n, d = x.shape
    e_per, _, h = expert_W.shape
    n_exp = router_W.shape[1]
    m_c = n // N_DEV

    def body(x_ref, rw_ref, idx_ref, ew_ref, sw_ref, out_ref,
             send_ref, recv_ref, send_sems, recv_sems):
        my = lax.axis_index("i")
        left = lax.rem(my + N_DEV - 1, N_DEV)
        right = lax.rem(my + 1, N_DEV)

        barrier_sem = pltpu.get_barrier_semaphore()
        for nbr in (left, right):
            pl.semaphore_signal(
                barrier_sem, inc=1,
                device_id=(nbr,), device_id_type=pl.DeviceIdType.MESH,
            )
        pl.semaphore_wait(barrier_sem, 2)

        xf = x_ref[...]
        scores = jnp.dot(xf, rw_ref[...], preferred_element_type=jnp.float32)
        s_max = jnp.max(scores, axis=-1, keepdims=True)
        e = jnp.exp(scores - s_max)
        probs = e / jnp.sum(e, axis=-1, keepdims=True)
        idx = idx_ref[...]
        eids = lax.broadcasted_iota(jnp.int32, (n, n_exp), 1)
        p_sel = jnp.sum(jnp.where(eids == idx, probs, 0.0), axis=-1,
                        keepdims=True)

        acc = jnp.zeros((n, h), jnp.float32)
        for k in range(e_per):
            w = jnp.where(idx == my * e_per + k, p_sel, 0.0)
            xk = (xf * w).astype(jnp.bfloat16)
            acc = acc + jnp.dot(xk, ew_ref[k].astype(jnp.bfloat16),
                                preferred_element_type=jnp.float32)
        out_ref[...] = acc

        for g in range(2 * (N_DEV - 1)):
            slot = g % 2
            sc = lax.rem(my + 2 * N_DEV - g, N_DEV) * m_c
            rc = lax.rem(my + 2 * N_DEV - g - 1, N_DEV) * m_c
            send_ref[slot, :, :] = out_ref[pl.ds(sc, m_c), :].astype(
                jnp.bfloat16)
            rdma = pltpu.make_async_remote_copy(
                src_ref=send_ref.at[slot],
                dst_ref=recv_ref.at[slot],
                send_sem=send_sems.at[slot],
                recv_sem=recv_sems.at[slot],
                device_id=(right,),
                device_id_type=pl.DeviceIdType.MESH,
            )
            rdma.start()
            rdma.wait()
            chunk = recv_ref[slot, :, :].astype(jnp.float32)
            if g < N_DEV - 1:
                out_ref[pl.ds(rc, m_c), :] = out_ref[pl.ds(rc, m_c), :] + chunk
            else:
                out_ref[pl.ds(rc, m_c), :] = chunk

        out_ref[...] = out_ref[...] + jnp.dot(
            xf.astype(jnp.bfloat16), sw_ref[...].astype(jnp.bfloat16),
            preferred_element_type=jnp.float32)

    return pl.pallas_call(
        body,
        out_shape=jax.ShapeDtypeStruct((n, h), jnp.float32),
        in_specs=[pl.BlockSpec(memory_space=pltpu.VMEM)] * 5,
        out_specs=pl.BlockSpec(memory_space=pltpu.VMEM),
        scratch_shapes=[
            pltpu.VMEM((2, m_c, h), jnp.bfloat16),
            pltpu.VMEM((2, m_c, h), jnp.bfloat16),
            pltpu.SemaphoreType.DMA((2,)),
            pltpu.SemaphoreType.DMA((2,)),
        ],
        compiler_params=pltpu.CompilerParams(
            collective_id=0,
            vmem_limit_bytes=96 * 1024 * 1024,
        ),
    )(x, router_W, route_idx, expert_W, shared_W)


# device time: 174289 ns/iter; 4.2949x vs baseline; 1.0190x over previous
import jax
import jax.numpy as jnp
from jax import lax
from jax.experimental import pallas as pl
from jax.experimental.pallas import tpu as pltpu

N_DEV = 16


def kernel(x, router_W, route_idx, expert_W, shared_W):
    n, d = x.shape
    e_per, _, h = expert_W.shape
    n_exp = router_W.shape[1]
    half = n // 2
    m_s = half // N_DEV

    def body(x_ref, rw_ref, idx_ref, ew_ref, sw_ref, out_ref,
             sbuf_r, sbuf_l, rbuf_r, rbuf_l,
             ssem_r, ssem_l, rsem_r, rsem_l):
        my = lax.axis_index("i")
        left = lax.rem(my + N_DEV - 1, N_DEV)
        right = lax.rem(my + 1, N_DEV)

        barrier_sem = pltpu.get_barrier_semaphore()
        for nbr in (left, right):
            pl.semaphore_signal(
                barrier_sem, inc=1,
                device_id=(nbr,), device_id_type=pl.DeviceIdType.MESH,
            )
        pl.semaphore_wait(barrier_sem, 2)

        xf = x_ref[...]
        scores = jnp.dot(xf, rw_ref[...], preferred_element_type=jnp.float32)
        s_max = jnp.max(scores, axis=-1, keepdims=True)
        e = jnp.exp(scores - s_max)
        probs = e / jnp.sum(e, axis=-1, keepdims=True)
        idx = idx_ref[...]
        eids = lax.broadcasted_iota(jnp.int32, (n, n_exp), 1)
        p_sel = jnp.sum(jnp.where(eids == idx, probs, 0.0), axis=-1,
                        keepdims=True)

        acc = jnp.zeros((n, h), jnp.float32)
        for k in range(e_per):
            w = jnp.where(idx == my * e_per + k, p_sel, 0.0)
            xk = (xf * w).astype(jnp.bfloat16)
            acc = acc + jnp.dot(xk, ew_ref[k].astype(jnp.bfloat16),
                                preferred_element_type=jnp.float32)
        out_ref[...] = acc

        streams = [
            dict(base=0, to=right,
                 cfun=lambda g: lax.rem(my + 2 * N_DEV - g, N_DEV),
                 sbuf=sbuf_r, rbuf=rbuf_r, ssem=ssem_r, rsem=rsem_r),
            dict(base=half, to=left,
                 cfun=lambda g: lax.rem(my + g, N_DEV),
                 sbuf=sbuf_l, rbuf=rbuf_l, ssem=ssem_l, rsem=rsem_l),
        ]

        n_hops = 2 * (N_DEV - 1)
        rdmas = [None, None]
        for g in range(n_hops):
            s2, s3, p3 = g % 2, g % 3, (g - 1) % 3
            for si, st in enumerate(streams):
                row0 = st["base"] + st["cfun"](g) * m_s
                if g == 0:
                    st["sbuf"][0, :, :] = out_ref[pl.ds(row0, m_s), :].astype(
                        jnp.bfloat16)
                    src = st["sbuf"].at[0]
                elif g <= N_DEV - 1:
                    comb = (out_ref[pl.ds(row0, m_s), :]
                            + st["rbuf"][p3, :, :].astype(jnp.float32))
                    st["sbuf"][s2, :, :] = comb.astype(jnp.bfloat16)
                    src = st["sbuf"].at[s2]
                else:
                    src = st["rbuf"].at[p3]
                rdma = pltpu.make_async_remote_copy(
                    src_ref=src,
                    dst_ref=st["rbuf"].at[s3],
                    send_sem=st["ssem"].at[s2],
                    recv_sem=st["rsem"].at[s3],
                    device_id=(st["to"],),
                    device_id_type=pl.DeviceIdType.MESH,
                )
                rdma.start()
                rdmas[si] = rdma
            if g == N_DEV - 1:
                for st in streams:
                    row0 = st["base"] + st["cfun"](g) * m_s
                    out_ref[pl.ds(row0, m_s), :] = (
                        out_ref[pl.ds(row0, m_s), :]
                        + st["rbuf"][p3, :, :].astype(jnp.float32))
            elif g > N_DEV - 1:
                for st in streams:
                    row0 = st["base"] + st["cfun"](g) * m_s
                    out_ref[pl.ds(row0, m_s), :] = st["rbuf"][p3, :, :].astype(
                        jnp.float32)
            for r in rdmas:
                r.wait()
        p3 = (n_hops - 1) % 3
        for st in streams:
            row0 = st["base"] + st["cfun"](n_hops) * m_s
            out_ref[pl.ds(row0, m_s), :] = st["rbuf"][p3, :, :].astype(
                jnp.float32)

        out_ref[...] = out_ref[...] + jnp.dot(
            xf.astype(jnp.bfloat16), sw_ref[...].astype(jnp.bfloat16),
            preferred_element_type=jnp.float32)

    return pl.pallas_call(
        body,
        out_shape=jax.ShapeDtypeStruct((n, h), jnp.float32),
        in_specs=[pl.BlockSpec(memory_space=pltpu.VMEM)] * 5,
        out_specs=pl.BlockSpec(memory_space=pltpu.VMEM),
        scratch_shapes=[
            pltpu.VMEM((2, m_s, h), jnp.bfloat16),
            pltpu.VMEM((2, m_s, h), jnp.bfloat16),
            pltpu.VMEM((3, m_s, h), jnp.bfloat16),
            pltpu.VMEM((3, m_s, h), jnp.bfloat16),
            pltpu.SemaphoreType.DMA((2,)),
            pltpu.SemaphoreType.DMA((2,)),
            pltpu.SemaphoreType.DMA((3,)),
            pltpu.SemaphoreType.DMA((3,)),
        ],
        compiler_params=pltpu.CompilerParams(
            collective_id=0,
            vmem_limit_bytes=96 * 1024 * 1024,
        ),
    )(x, router_W, route_idx, expert_W, shared_W)
